# baseline (speedup 1.0000x reference)
import functools

import jax
import jax.numpy as jnp
from jax import lax
from jax.experimental import pallas as pl
from jax.experimental.pallas import tpu as pltpu
from jax.experimental.pallas import tpu_sc as plsc

N = 10000
C = 128
NC = 2
NS = 16
NW = NC * NS
NP = 10112
ROWS_PER_TILE = NP // NS
K = 128
E = 320000
NCH0 = 54
NCH1 = 104
MAXC = max(NCH0, NCH1)
EPAD = NS * (NCH0 + NCH1) * K

_mesh = plsc.VectorSubcoreMesh(
    core_axis_name="c", subcore_axis_name="s", num_cores=NC, num_subcores=NS)


@functools.partial(
    pl.kernel,
    mesh=_mesh,
    compiler_params=pltpu.CompilerParams(use_tc_tiling_on_sc=False),
    out_type=jax.ShapeDtypeStruct((NC, NP, C), jnp.float32),
    scratch_types=[
        pltpu.VMEM((2, K), jnp.int32),
        pltpu.VMEM((2, K), jnp.int32),
        pltpu.VMEM((2, K, C), jnp.float32),
        pltpu.VMEM_SHARED((NP, C), jnp.float32),
        pltpu.SemaphoreType.DMA,
        pltpu.SemaphoreType.DMA,
    ],
)
def _mp_sum_sc(xt_hbm, zeros_hbm, src_hbm, dst_hbm, out_hbm,
               src_v, dst_v, rows_v, acc_s, sem, semi):
    cid = lax.axis_index("c")
    sid = lax.axis_index("s")
    w = cid * NS + sid
    base = sid * ROWS_PER_TILE
    nch = lax.select(cid == 0, NCH0, NCH1)

    @pl.when(cid == 0)
    def _():
        pltpu.sync_copy(xt_hbm.at[pl.ds(base, ROWS_PER_TILE)],
                        acc_s.at[pl.ds(base, ROWS_PER_TILE)])

    @pl.when(cid != 0)
    def _():
        pltpu.sync_copy(zeros_hbm.at[pl.ds(base, ROWS_PER_TILE)],
                        acc_s.at[pl.ds(base, ROWS_PER_TILE)])

    plsc.subcore_barrier()

    pltpu.sync_copy(src_hbm.at[w, 0], src_v.at[0])
    pltpu.sync_copy(dst_hbm.at[w, 0], dst_v.at[0])
    pltpu.async_copy(xt_hbm.at[src_v.at[0]], rows_v.at[0], sem)
    pltpu.async_copy(src_hbm.at[w, 1], src_v.at[1], semi)
    pltpu.async_copy(dst_hbm.at[w, 1], dst_v.at[1], semi)

    def body(j2, carry):
        for b in (0, 1):
            j = 2 * j2 + b
            pltpu.make_async_copy(xt_hbm.at[src_v.at[b]],
                                  rows_v.at[b], sem).wait()

            @pl.when(j + 1 < nch)
            def _():
                pltpu.make_async_copy(src_hbm.at[w, j + 1],
                                      src_v.at[1 - b], semi).wait()
                pltpu.make_async_copy(dst_hbm.at[w, j + 1],
                                      dst_v.at[1 - b], semi).wait()
                pltpu.async_copy(xt_hbm.at[src_v.at[1 - b]],
                                 rows_v.at[1 - b], sem)

            pltpu.sync_copy(rows_v.at[b], acc_s.at[dst_v.at[b]], add=True)

            @pl.when(j + 2 < nch)
            def _():
                pltpu.async_copy(src_hbm.at[w, j + 2], src_v.at[b], semi)
                pltpu.async_copy(dst_hbm.at[w, j + 2], dst_v.at[b], semi)

        return carry

    lax.fori_loop(0, nch // 2, body, 0)

    plsc.subcore_barrier()

    pltpu.sync_copy(acc_s.at[pl.ds(base, ROWS_PER_TILE)],
                    out_hbm.at[cid, pl.ds(base, ROWS_PER_TILE)])


def _pack_idx(flat):
    n0 = NS * NCH0 * K
    e0 = flat[:n0].reshape(NS, NCH0, K)
    e0 = jnp.pad(e0, ((0, 0), (0, MAXC - NCH0), (0, 0)), constant_values=N)
    e1 = flat[n0:].reshape(NS, NCH1, K)
    e1 = jnp.pad(e1, ((0, 0), (0, MAXC - NCH1), (0, 0)), constant_values=N)
    return jnp.concatenate([e0, e1], axis=0)


def kernel(x, edge_index):
    xt = jnp.transpose(x.reshape(C, N))
    xt = jnp.pad(xt, ((0, NP - N), (0, 0)))
    zeros = jnp.zeros((NP, C), jnp.float32)

    src = edge_index[0].astype(jnp.int32)
    dst = edge_index[1].astype(jnp.int32)
    pad = jnp.full((EPAD - E,), N, jnp.int32)
    src = _pack_idx(jnp.concatenate([src, pad]))
    dst = _pack_idx(jnp.concatenate([dst, pad]))

    partial_tables = _mp_sum_sc(xt, zeros, src, dst)
    out = partial_tables[0, :N] + partial_tables[1, :N]
    return jnp.transpose(out).reshape(1, C, N, 1)


# speedup vs baseline: 1.2204x; 1.2204x over previous
import functools

import jax
import jax.numpy as jnp
from jax import lax
from jax.experimental import pallas as pl
from jax.experimental.pallas import tpu as pltpu
from jax.experimental.pallas import tpu_sc as plsc

N = 10000
C = 128
NC = 2
NS = 16
NW = NC * NS
NP = 10112
ROWS_PER_TILE = NP // NS
K = 128
E = 320000
NCH0 = 104
NCH1 = 54
MAXC = max(NCH0, NCH1)
EPAD = NS * (NCH0 + NCH1) * K

_mesh = plsc.VectorSubcoreMesh(
    core_axis_name="c", subcore_axis_name="s", num_cores=NC, num_subcores=NS)


@functools.partial(
    pl.kernel,
    mesh=_mesh,
    compiler_params=pltpu.CompilerParams(use_tc_tiling_on_sc=False),
    out_type=jax.ShapeDtypeStruct((NC, NP, C), jnp.float32),
    scratch_types=[
        pltpu.VMEM((2, K), jnp.int32),
        pltpu.VMEM((2, K), jnp.int32),
        pltpu.VMEM((2, K, C), jnp.float32),
        pltpu.VMEM_SHARED((NP, C), jnp.float32),
        pltpu.SemaphoreType.DMA,
        pltpu.SemaphoreType.DMA,
    ],
)
def _mp_sum_sc(xt_hbm, zeros_hbm, src_hbm, dst_hbm, out_hbm,
               src_v, dst_v, rows_v, acc_s, sem, semi):
    cid = lax.axis_index("c")
    sid = lax.axis_index("s")
    w = cid * NS + sid
    base = sid * ROWS_PER_TILE
    nch = lax.select(cid == 0, NCH0, NCH1)

    @pl.when(cid == 0)
    def _():
        pltpu.sync_copy(xt_hbm.at[pl.ds(base, ROWS_PER_TILE)],
                        acc_s.at[pl.ds(base, ROWS_PER_TILE)])

    @pl.when(cid != 0)
    def _():
        pltpu.sync_copy(zeros_hbm.at[pl.ds(base, ROWS_PER_TILE)],
                        acc_s.at[pl.ds(base, ROWS_PER_TILE)])

    plsc.subcore_barrier()

    pltpu.sync_copy(src_hbm.at[w, 0], src_v.at[0])
    pltpu.sync_copy(dst_hbm.at[w, 0], dst_v.at[0])
    pltpu.async_copy(xt_hbm.at[src_v.at[0]], rows_v.at[0], sem)
    pltpu.async_copy(src_hbm.at[w, 1], src_v.at[1], semi)
    pltpu.async_copy(dst_hbm.at[w, 1], dst_v.at[1], semi)

    def body(j2, carry):
        for b in (0, 1):
            j = 2 * j2 + b
            pltpu.make_async_copy(xt_hbm.at[src_v.at[b]],
                                  rows_v.at[b], sem).wait()

            @pl.when(j + 1 < nch)
            def _():
                pltpu.make_async_copy(src_hbm.at[w, j + 1],
                                      src_v.at[1 - b], semi).wait()
                pltpu.make_async_copy(dst_hbm.at[w, j + 1],
                                      dst_v.at[1 - b], semi).wait()
                pltpu.async_copy(xt_hbm.at[src_v.at[1 - b]],
                                 rows_v.at[1 - b], sem)

            pltpu.sync_copy(rows_v.at[b], acc_s.at[dst_v.at[b]], add=True)

            @pl.when(j + 2 < nch)
            def _():
                pltpu.async_copy(src_hbm.at[w, j + 2], src_v.at[b], semi)
                pltpu.async_copy(dst_hbm.at[w, j + 2], dst_v.at[b], semi)

        return carry

    lax.fori_loop(0, nch // 2, body, 0)

    plsc.subcore_barrier()

    pltpu.sync_copy(acc_s.at[pl.ds(base, ROWS_PER_TILE)],
                    out_hbm.at[cid, pl.ds(base, ROWS_PER_TILE)])


def _pack_idx(flat):
    n0 = NS * NCH0 * K
    e0 = flat[:n0].reshape(NS, NCH0, K)
    e0 = jnp.pad(e0, ((0, 0), (0, MAXC - NCH0), (0, 0)), constant_values=N)
    e1 = flat[n0:].reshape(NS, NCH1, K)
    e1 = jnp.pad(e1, ((0, 0), (0, MAXC - NCH1), (0, 0)), constant_values=N)
    return jnp.concatenate([e0, e1], axis=0)


def kernel(x, edge_index):
    xt = jnp.transpose(x.reshape(C, N))
    xt = jnp.pad(xt, ((0, NP - N), (0, 0)))
    zeros = jnp.zeros((NP, C), jnp.float32)

    src = edge_index[0].astype(jnp.int32)
    dst = edge_index[1].astype(jnp.int32)
    pad = jnp.full((EPAD - E,), N, jnp.int32)
    src = _pack_idx(jnp.concatenate([src, pad]))
    dst = _pack_idx(jnp.concatenate([dst, pad]))

    partial_tables = _mp_sum_sc(xt, zeros, src, dst)
    out = partial_tables[0, :N] + partial_tables[1, :N]
    return jnp.transpose(out).reshape(1, C, N, 1)
